# baseline (speedup 1.0000x reference)
import functools

import jax
import jax.numpy as jnp
from jax import lax
from jax.experimental import pallas as pl
from jax.experimental.pallas import tpu as pltpu
from jax.experimental.pallas import tpu_sc as plsc

_NC = 2
_NS = 16
_NW = _NC * _NS
_WAVE = 16


def kernel(users, pos_items, neg_items, user_table, item_table):
    B = users.shape[0]
    D = user_table.shape[1]
    b_per_w = B // _NW

    u_idx = users.astype(jnp.int32)
    p_idx = pos_items.astype(jnp.int32)
    n_idx = neg_items.astype(jnp.int32)
    tu = user_table.T
    ti = item_table.T

    mesh = plsc.VectorSubcoreMesh(core_axis_name="c", subcore_axis_name="s")
    out_sds = jax.ShapeDtypeStruct((D, B), jnp.float32)

    @functools.partial(
        pl.kernel,
        mesh=mesh,
        compiler_params=pltpu.CompilerParams(use_tc_tiling_on_sc=False),
        out_type=(out_sds, out_sds, out_sds),
        scratch_types=[
            pltpu.VMEM((b_per_w,), jnp.int32),
            pltpu.VMEM((D, b_per_w), jnp.float32),
            pltpu.SemaphoreType.DMA,
        ],
    )
    def gather_kernel(tu_hbm, ui_hbm, pi_hbm, ni_hbm, ti_hbm,
                      ou_hbm, op_hbm, on_hbm,
                      idx_v, dst_v, sem):
        wid = lax.axis_index("s") * _NC + lax.axis_index("c")
        base = wid * b_per_w

        for tbl, idx_hbm, out_hbm in (
            (tu_hbm, ui_hbm, ou_hbm),
            (ti_hbm, pi_hbm, op_hbm),
            (ti_hbm, ni_hbm, on_hbm),
        ):
            pltpu.sync_copy(idx_hbm.at[pl.ds(base, b_per_w)], idx_v)

            @pl.loop(0, D, step=_WAVE)
            def _(c0):
                @pl.loop(0, _WAVE)
                def _(j):
                    c = c0 + j
                    pltpu.make_async_copy(
                        tbl.at[c].at[idx_v], dst_v.at[c], sem
                    ).start()

                @pl.loop(0, _WAVE)
                def _(j):
                    c = c0 + j
                    pltpu.make_async_copy(
                        tbl.at[c].at[idx_v], dst_v.at[c], sem
                    ).wait()

            pltpu.sync_copy(dst_v, out_hbm.at[:, pl.ds(base, b_per_w)])

    ou, op, on = gather_kernel(tu, u_idx, p_idx, n_idx, ti)
    return (ou.T, op.T, on.T)


# speedup vs baseline: 11.9249x; 11.9249x over previous
import functools

import jax
import jax.numpy as jnp
from jax import lax
from jax.experimental import pallas as pl
from jax.experimental.pallas import tpu as pltpu
from jax.experimental.pallas import tpu_sc as plsc

_NC = 2
_NS = 16
_NW = _NC * _NS
_K = 16


def kernel(users, pos_items, neg_items, user_table, item_table):
    B = users.shape[0]
    N, D = user_table.shape
    b_per_w = B // _NW

    u_idx = users.astype(jnp.int32)
    p_idx = pos_items.astype(jnp.int32)
    n_idx = neg_items.astype(jnp.int32)

    mesh = plsc.VectorSubcoreMesh(core_axis_name="c", subcore_axis_name="s")
    out_sds = jax.ShapeDtypeStruct((B, D), jnp.float32)

    @functools.partial(
        pl.kernel,
        mesh=mesh,
        out_type=(out_sds, out_sds, out_sds),
        scratch_types=[
            pltpu.VMEM((b_per_w,), jnp.int32),
            pltpu.VMEM((_K, 8, D), jnp.float32),
            pltpu.VMEM((b_per_w, D), jnp.float32),
            pltpu.SemaphoreType.DMA,
        ],
    )
    def gather_kernel(tu_hbm, ui_hbm, pi_hbm, ni_hbm, ti_hbm,
                      ou_hbm, op_hbm, on_hbm,
                      idx_v, grp_v, dst_v, sem):
        wid = lax.axis_index("s") * _NC + lax.axis_index("c")
        base = wid * b_per_w

        for tbl, idx_hbm, out_hbm in (
            (tu_hbm, ui_hbm, ou_hbm),
            (ti_hbm, pi_hbm, op_hbm),
            (ti_hbm, ni_hbm, on_hbm),
        ):
            pltpu.sync_copy(idx_hbm.at[pl.ds(base, b_per_w)], idx_v)

            @pl.loop(0, b_per_w, step=_K)
            def _(w):
                vec = idx_v[pl.ds(w, _K)]
                for s in range(_K):
                    g = (vec[s] // 8) * 8
                    pltpu.make_async_copy(
                        tbl.at[pl.ds(g, 8)], grp_v.at[s], sem
                    ).start()
                for s in range(_K):
                    r = vec[s]
                    g = (r // 8) * 8
                    pltpu.make_async_copy(
                        tbl.at[pl.ds(g, 8)], grp_v.at[s], sem
                    ).wait()
                    rm = r - g
                    for c0 in range(0, D, 16):
                        dst_v[w + s, pl.ds(c0, 16)] = grp_v[s, rm, pl.ds(c0, 16)]

            pltpu.sync_copy(dst_v, out_hbm.at[pl.ds(base, b_per_w)])

    return gather_kernel(user_table, u_idx, p_idx, n_idx, item_table)


# speedup vs baseline: 12.4053x; 1.0403x over previous
import functools

import jax
import jax.numpy as jnp
from jax import lax
from jax.experimental import pallas as pl
from jax.experimental.pallas import tpu as pltpu
from jax.experimental.pallas import tpu_sc as plsc

_NC = 2
_NS = 16
_NW = _NC * _NS
_K = 32


def kernel(users, pos_items, neg_items, user_table, item_table):
    B = users.shape[0]
    N, D = user_table.shape
    b_per_w = B // _NW

    u_idx = users.astype(jnp.int32)
    p_idx = pos_items.astype(jnp.int32)
    n_idx = neg_items.astype(jnp.int32)

    mesh = plsc.VectorSubcoreMesh(core_axis_name="c", subcore_axis_name="s")
    out_sds = jax.ShapeDtypeStruct((B, D), jnp.float32)

    @functools.partial(
        pl.kernel,
        mesh=mesh,
        out_type=(out_sds, out_sds, out_sds),
        scratch_types=[
            pltpu.VMEM((b_per_w,), jnp.int32),
            pltpu.VMEM((_K, 8, D), jnp.float32),
            pltpu.VMEM((b_per_w, D), jnp.float32),
            pltpu.SemaphoreType.DMA,
        ],
    )
    def gather_kernel(tu_hbm, ui_hbm, pi_hbm, ni_hbm, ti_hbm,
                      ou_hbm, op_hbm, on_hbm,
                      idx_v, grp_v, dst_v, sem):
        wid = lax.axis_index("s") * _NC + lax.axis_index("c")
        base = wid * b_per_w

        for tbl, idx_hbm, out_hbm in (
            (tu_hbm, ui_hbm, ou_hbm),
            (ti_hbm, pi_hbm, op_hbm),
            (ti_hbm, ni_hbm, on_hbm),
        ):
            pltpu.sync_copy(idx_hbm.at[pl.ds(base, b_per_w)], idx_v)

            @pl.loop(0, b_per_w, step=_K)
            def _(w):
                vecs = [idx_v[pl.ds(w + 16 * h, 16)] for h in range(_K // 16)]
                for s in range(_K):
                    g = (vecs[s // 16][s % 16] // 8) * 8
                    pltpu.make_async_copy(
                        tbl.at[pl.ds(g, 8)], grp_v.at[s], sem
                    ).start()
                for s in range(_K):
                    r = vecs[s // 16][s % 16]
                    g = (r // 8) * 8
                    pltpu.make_async_copy(
                        tbl.at[pl.ds(g, 8)], grp_v.at[s], sem
                    ).wait()
                    rm = r - g
                    for c0 in range(0, D, 16):
                        dst_v[w + s, pl.ds(c0, 16)] = grp_v[s, rm, pl.ds(c0, 16)]

            pltpu.sync_copy(dst_v, out_hbm.at[pl.ds(base, b_per_w)])

    return gather_kernel(user_table, u_idx, p_idx, n_idx, item_table)
